# baseline (speedup 1.0000x reference)
import functools

import jax
import jax.numpy as jnp
from jax.experimental import pallas as pl
from jax.experimental.pallas import tpu as pltpu


def _se_kernel(x_ref, w1_ref, b1_ref, w2_ref, b2_ref, o_ref, *, s_len):
    xb = x_ref[0]
    inv = 1.0 / s_len
    lo = (s_len // 128) * 128
    hi = lo + 128

    left = xb[:, :lo]
    mid = xb[:, lo:hi]
    right = xb[:, hi:]

    s_l = jnp.sum(left, axis=1, keepdims=True)
    s_r = jnp.sum(right, axis=1, keepdims=True)
    mm = jax.lax.broadcasted_iota(jnp.int32, (128, 128), 1) < (s_len - lo)
    s_m_e = jnp.sum(jnp.where(mm, mid, 0.0), axis=1, keepdims=True)
    s_m = jnp.sum(mid, axis=1, keepdims=True)

    s_even = (s_l + s_m_e) * inv
    s_odd = (s_r + s_m - s_m_e) * inv
    s_perm = jnp.concatenate([s_even, s_odd], axis=0)

    h = jnp.dot(w1_ref[...], s_perm, preferred_element_type=jnp.float32)
    h = jnp.maximum(h + b1_ref[...], 0.0)
    g = jnp.dot(w2_ref[...], h, preferred_element_type=jnp.float32)
    g = jax.nn.sigmoid(g + b2_ref[...])
    ge = g[:128, :]
    go = g[128:, :]

    o_ref[0, :, :lo] = left * ge
    o_ref[0, :, lo:hi] = mid * jnp.where(mm, ge, go)
    o_ref[0, :, hi:] = right * go


def kernel(x, w1, b1, w2, b2):
    B, C, H, W = x.shape
    S = H * W
    BOT = w1.shape[0]
    R = C // 2
    xr = x.reshape(B, R, 2 * S)

    perm = jnp.concatenate([jnp.arange(0, C, 2), jnp.arange(1, C, 2)])
    w1p = w1[:, perm]
    w2p = w2[perm, :]
    b1c = b1.reshape(BOT, 1)
    b2p = b2[perm].reshape(C, 1)

    body = functools.partial(_se_kernel, s_len=S)

    out = pl.pallas_call(
        body,
        grid=(B,),
        in_specs=[
            pl.BlockSpec((1, R, 2 * S), lambda b: (b, 0, 0)),
            pl.BlockSpec((BOT, C), lambda b: (0, 0)),
            pl.BlockSpec((BOT, 1), lambda b: (0, 0)),
            pl.BlockSpec((C, BOT), lambda b: (0, 0)),
            pl.BlockSpec((C, 1), lambda b: (0, 0)),
        ],
        out_specs=pl.BlockSpec((1, R, 2 * S), lambda b: (b, 0, 0)),
        out_shape=jax.ShapeDtypeStruct((B, R, 2 * S), jnp.float32),
        compiler_params=pltpu.CompilerParams(
            dimension_semantics=("parallel",),
        ),
    )(xr, w1p, b1c, w2p, b2p)
    return out.reshape(B, C, H, W)


# speedup vs baseline: 1.4243x; 1.4243x over previous
import functools

import jax
import jax.numpy as jnp
from jax.experimental import pallas as pl
from jax.experimental.pallas import tpu as pltpu


def _se_kernel(x_ref, w1_ref, b1_ref, w2_ref, b2_ref, o_ref, *, inv_hw):
    xb = x_ref[0]
    s = jnp.sum(xb, axis=(1, 2))[:, None] * inv_hw
    h = jnp.dot(w1_ref[...], s, preferred_element_type=jnp.float32)
    h = jnp.maximum(h + b1_ref[...], 0.0)
    g = jnp.dot(w2_ref[...], h, preferred_element_type=jnp.float32)
    g = jax.nn.sigmoid(g + b2_ref[...])
    o_ref[0] = xb * g[:, :, None]


def kernel(x, w1, b1, w2, b2):
    B, C, H, W = x.shape
    BOT = w1.shape[0]
    b1c = b1.reshape(BOT, 1)
    b2c = b2.reshape(C, 1)

    body = functools.partial(_se_kernel, inv_hw=1.0 / (H * W))

    return pl.pallas_call(
        body,
        grid=(B,),
        in_specs=[
            pl.BlockSpec((1, C, H, W), lambda b: (b, 0, 0, 0)),
            pl.BlockSpec((BOT, C), lambda b: (0, 0)),
            pl.BlockSpec((BOT, 1), lambda b: (0, 0)),
            pl.BlockSpec((C, BOT), lambda b: (0, 0)),
            pl.BlockSpec((C, 1), lambda b: (0, 0)),
        ],
        out_specs=pl.BlockSpec((1, C, H, W), lambda b: (b, 0, 0, 0)),
        out_shape=jax.ShapeDtypeStruct((B, C, H, W), jnp.float32),
        compiler_params=pltpu.CompilerParams(
            dimension_semantics=("parallel",),
        ),
    )(x, w1, b1c, w2, b2c)


# speedup vs baseline: 9.2631x; 6.5036x over previous
import functools

import jax
import jax.numpy as jnp
from jax.experimental import pallas as pl
from jax.experimental.pallas import tpu as pltpu


def _se_kernel(x_ref, w1_ref, b1_ref, w2_ref, b2_ref, o_ref, *, inv_hw):
    xb = x_ref[0]
    s = jnp.sum(xb, axis=(0, 1))[None, :] * inv_hw
    h = jnp.dot(s, w1_ref[...], preferred_element_type=jnp.float32)
    h = jnp.maximum(h + b1_ref[...], 0.0)
    g = jnp.dot(h, w2_ref[...], preferred_element_type=jnp.float32)
    g = jax.nn.sigmoid(g + b2_ref[...])
    o_ref[0] = xb * g[None, :, :]


def kernel(x, w1, b1, w2, b2):
    B, C, H, W = x.shape
    BOT = w1.shape[0]
    xt = x.transpose(0, 2, 3, 1)
    w1t = w1.T
    w2t = w2.T
    b1r = b1.reshape(1, BOT)
    b2r = b2.reshape(1, C)

    body = functools.partial(_se_kernel, inv_hw=1.0 / (H * W))

    out = pl.pallas_call(
        body,
        grid=(B,),
        in_specs=[
            pl.BlockSpec((1, H, W, C), lambda b: (b, 0, 0, 0)),
            pl.BlockSpec((C, BOT), lambda b: (0, 0)),
            pl.BlockSpec((1, BOT), lambda b: (0, 0)),
            pl.BlockSpec((BOT, C), lambda b: (0, 0)),
            pl.BlockSpec((1, C), lambda b: (0, 0)),
        ],
        out_specs=pl.BlockSpec((1, H, W, C), lambda b: (b, 0, 0, 0)),
        out_shape=jax.ShapeDtypeStruct((B, H, W, C), jnp.float32),
        compiler_params=pltpu.CompilerParams(
            dimension_semantics=("parallel",),
        ),
    )(xt, w1t, b1r, w2t, b2r)
    return out.transpose(0, 3, 1, 2)


# speedup vs baseline: 9.9658x; 1.0759x over previous
import functools

import jax
import jax.numpy as jnp
from jax.experimental import pallas as pl
from jax.experimental.pallas import tpu as pltpu


def _se_kernel(x_ref, w1_ref, b1_ref, w2_ref, b2_ref, o_ref, *, inv_hw):
    xb = x_ref[...]
    s = jnp.sum(xb, axis=(1, 2)) * inv_hw
    h = jnp.dot(s, w1_ref[...], preferred_element_type=jnp.float32)
    h = jnp.maximum(h + b1_ref[...], 0.0)
    g = jnp.dot(h, w2_ref[...], preferred_element_type=jnp.float32)
    g = jax.nn.sigmoid(g + b2_ref[...])
    o_ref[...] = xb * g[:, None, None, :]


def kernel(x, w1, b1, w2, b2):
    B, C, H, W = x.shape
    BOT = w1.shape[0]
    xt = x.transpose(0, 2, 3, 1)
    w1t = w1.T
    w2t = w2.T
    b1r = b1.reshape(1, BOT)
    b2r = b2.reshape(1, C)

    body = functools.partial(_se_kernel, inv_hw=1.0 / (H * W))

    BSZ = 2
    out = pl.pallas_call(
        body,
        grid=(B // BSZ,),
        in_specs=[
            pl.BlockSpec((BSZ, H, W, C), lambda b: (b, 0, 0, 0)),
            pl.BlockSpec((C, BOT), lambda b: (0, 0)),
            pl.BlockSpec((1, BOT), lambda b: (0, 0)),
            pl.BlockSpec((BOT, C), lambda b: (0, 0)),
            pl.BlockSpec((1, C), lambda b: (0, 0)),
        ],
        out_specs=pl.BlockSpec((BSZ, H, W, C), lambda b: (b, 0, 0, 0)),
        out_shape=jax.ShapeDtypeStruct((B, H, W, C), jnp.float32),
        compiler_params=pltpu.CompilerParams(
            dimension_semantics=("parallel",),
        ),
    )(xt, w1t, b1r, w2t, b2r)
    return out.transpose(0, 3, 1, 2)


# speedup vs baseline: 10.2432x; 1.0278x over previous
import functools

import jax
import jax.numpy as jnp
from jax.experimental import pallas as pl
from jax.experimental.pallas import tpu as pltpu


def _se_kernel(x_ref, w1_ref, b1_ref, w2_ref, b2_ref, o_ref, *, inv_hw):
    xb = x_ref[...]
    s = jnp.sum(xb, axis=(1, 2)) * inv_hw
    h = jnp.dot(s, w1_ref[...], preferred_element_type=jnp.float32)
    h = jnp.maximum(h + b1_ref[...], 0.0)
    g = jnp.dot(h, w2_ref[...], preferred_element_type=jnp.float32)
    g = jax.nn.sigmoid(g + b2_ref[...])
    o_ref[...] = xb * g[:, None, None, :]


def kernel(x, w1, b1, w2, b2):
    B, C, H, W = x.shape
    BOT = w1.shape[0]
    xt = x.transpose(0, 2, 3, 1)
    w1t = w1.T
    w2t = w2.T
    b1r = b1.reshape(1, BOT)
    b2r = b2.reshape(1, C)

    body = functools.partial(_se_kernel, inv_hw=1.0 / (H * W))

    BSZ = 4
    out = pl.pallas_call(
        body,
        grid=(B // BSZ,),
        in_specs=[
            pl.BlockSpec((BSZ, H, W, C), lambda b: (b, 0, 0, 0)),
            pl.BlockSpec((C, BOT), lambda b: (0, 0)),
            pl.BlockSpec((1, BOT), lambda b: (0, 0)),
            pl.BlockSpec((BOT, C), lambda b: (0, 0)),
            pl.BlockSpec((1, C), lambda b: (0, 0)),
        ],
        out_specs=pl.BlockSpec((BSZ, H, W, C), lambda b: (b, 0, 0, 0)),
        out_shape=jax.ShapeDtypeStruct((B, H, W, C), jnp.float32),
        compiler_params=pltpu.CompilerParams(
            dimension_semantics=("parallel",),
        ),
    )(xt, w1t, b1r, w2t, b2r)
    return out.transpose(0, 3, 1, 2)
